# baseline (speedup 1.0000x reference)
import functools

import jax
import jax.numpy as jnp
from jax import lax
from jax.experimental import pallas as pl
from jax.experimental.pallas import tpu as pltpu
from jax.experimental.pallas import tpu_sc as plsc

EMB = 16
NC = 2
NS = 16
NW = NC * NS
CHUNK = 1024
NBUF = 2


def _edge_encode(n_edges):
    n_chunks = n_edges // CHUNK
    full, extra = divmod(n_chunks, NW)

    mesh = plsc.VectorSubcoreMesh(core_axis_name="c", subcore_axis_name="s")

    @functools.partial(
        pl.kernel,
        mesh=mesh,
        out_type=jax.ShapeDtypeStruct((n_edges * EMB // 128, 128), jnp.float32),
        compiler_params=pltpu.CompilerParams(
            needs_layout_passes=False, use_tc_tiling_on_sc=False
        ),
        scratch_types=[
            pltpu.VMEM((NBUF, CHUNK, 2), jnp.int32),
            pltpu.VMEM((NBUF, CHUNK * EMB // 128, 128), jnp.float32),
            pltpu.VMEM((2, EMB), jnp.float32),
            pltpu.VMEM((2, EMB), jnp.float32),
            pltpu.VMEM((4 * EMB,), jnp.float32),
            pltpu.SemaphoreType.DMA,
            pltpu.SemaphoreType.DMA,
            pltpu.SemaphoreType.DMA,
            pltpu.SemaphoreType.DMA,
        ],
    )
    def run(attr_hbm, wt_hbm, wd_hbm, out_hbm,
            attr_v, rows_v, wt_v, wd_v, lut_v, si0, si1, so0, so1):
        wid = lax.axis_index("s") * NC + lax.axis_index("c")
        iota = lax.iota(jnp.int32, 16)
        zeros = jnp.zeros((16,), jnp.int32)
        sem_in = [si0, si1]
        sem_out = [so0, so1]

        pltpu.sync_copy(wt_hbm, wt_v)
        pltpu.sync_copy(wd_hbm, wd_v)
        wt0 = wt_v[0, :]
        wt1 = wt_v[1, :]
        wd0 = wd_v[0, :]
        wd1 = wd_v[1, :]
        lut_v[pl.ds(0, 16)] = wt0 + wd0
        lut_v[pl.ds(16, 16)] = wt0 + wd1
        lut_v[pl.ds(32, 16)] = wt1 + wd0
        lut_v[pl.ds(48, 16)] = wt1 + wd1

        n_mine = full + jnp.where(wid < extra, 1, 0)

        def in_copy(t, b):
            base = (wid + t * NW) * CHUNK
            return pltpu.make_async_copy(
                attr_hbm.at[pl.ds(base, CHUNK), :], attr_v.at[b], sem_in[b]
            )

        OROWS = CHUNK * EMB // 128

        def out_copy(t, b):
            base = (wid + t * NW) * OROWS
            return pltpu.make_async_copy(
                rows_v.at[b], out_hbm.at[pl.ds(base, OROWS), :], sem_out[b]
            )

        in_copy(0, 0).start()

        def compute(b):
            def group_body(g, c2):
                rows16 = g * 16 + iota
                a0 = plsc.load_gather(attr_v.at[b], [rows16, zeros])
                a1 = plsc.load_gather(attr_v.at[b], [rows16, zeros + 1])
                cb = (a0 * 2 + a1) * 16
                r = rows16 >> 3
                c0 = (rows16 & 7) * 16
                vals = []
                for d in range(16):
                    pm = jnp.bitwise_and(iota + d, 15)
                    vals.append(plsc.load_gather(lut_v, [cb + pm]))
                for d in range(16):
                    pm = jnp.bitwise_and(iota + d, 15)
                    plsc.store_scatter(rows_v.at[b], [r, c0 + pm], vals[d])
                return c2

            lax.fori_loop(0, CHUNK // 16, group_body, 0)

        def super_body(tt, carry):
            for b in range(NBUF):
                t = tt * NBUF + b

                @pl.when(t < n_mine)
                def _():
                    @pl.when(t + 1 < n_mine)
                    def _():
                        in_copy(t + 1, (b + 1) % NBUF).start()

                    in_copy(t, b).wait()

                    @pl.when(t >= NBUF)
                    def _():
                        out_copy(t - NBUF, b).wait()

                    compute(b)
                    out_copy(t, b).start()

            return carry

        n_super = (full + 1 + NBUF - 1) // NBUF
        lax.fori_loop(0, n_super, super_body, 0)

        for b in range(NBUF):
            @pl.when(n_mine > b)
            def _():
                t_last = ((n_mine - 1 - b) // NBUF) * NBUF + b
                out_copy(t_last, b).wait()

    return run


def _relayout_out(out128, n_edges):
    rows = 512

    def body(in_ref, out_ref):
        x = in_ref[...]
        for s in range(8):
            out_ref[s::8, :] = x[:, s * EMB:(s + 1) * EMB]

    return pl.pallas_call(
        body,
        grid=(n_edges // rows,),
        in_specs=[pl.BlockSpec((rows * EMB // 128, 128), lambda i: (i, 0))],
        out_specs=pl.BlockSpec((rows, EMB), lambda i: (i, 0)),
        out_shape=jax.ShapeDtypeStruct((n_edges, EMB), jnp.float32),
    )(out128)


def kernel(edge_attr, W_type, W_dir):
    n_edges = edge_attr.shape[0]
    run = _edge_encode(n_edges)
    out128 = run(edge_attr.astype(jnp.int32), W_type, W_dir)
    return _relayout_out(out128, n_edges)


# speedup vs baseline: 1.5840x; 1.5840x over previous
import functools

import jax
import jax.numpy as jnp
from jax import lax
from jax.experimental import pallas as pl
from jax.experimental.pallas import tpu as pltpu
from jax.experimental.pallas import tpu_sc as plsc

EMB = 16
NC = 2
NS = 16
NW = NC * NS


def _edge_encode(n_edges):
    per_w = n_edges // NW
    chunk = 2000
    while per_w % chunk:
        chunk -= 16
    n_chunks = per_w // chunk

    mesh = plsc.VectorSubcoreMesh(core_axis_name="c", subcore_axis_name="s")

    @functools.partial(
        pl.kernel,
        mesh=mesh,
        out_type=jax.ShapeDtypeStruct((n_edges * EMB,), jnp.float32),
        compiler_params=pltpu.CompilerParams(needs_layout_passes=False),
        scratch_types=[
            pltpu.VMEM((chunk * 2,), jnp.int32),
            pltpu.VMEM((chunk * EMB,), jnp.float32),
            pltpu.VMEM((2 * EMB,), jnp.float32),
            pltpu.VMEM((2 * EMB,), jnp.float32),
            pltpu.VMEM((4 * EMB,), jnp.float32),
        ],
    )
    def run(attr_hbm, wt_hbm, wd_hbm, out_hbm, attr_v, rows_v, wt_v, wd_v, lut_v):
        wid = lax.axis_index("s") * NC + lax.axis_index("c")
        iota = lax.iota(jnp.int32, 16)

        pltpu.sync_copy(wt_hbm, wt_v)
        pltpu.sync_copy(wd_hbm, wd_v)
        wt0 = wt_v[pl.ds(0, 16)]
        wt1 = wt_v[pl.ds(16, 16)]
        wd0 = wd_v[pl.ds(0, 16)]
        wd1 = wd_v[pl.ds(16, 16)]
        lut_v[pl.ds(0, 16)] = wt0 + wd0
        lut_v[pl.ds(16, 16)] = wt0 + wd1
        lut_v[pl.ds(32, 16)] = wt1 + wd0
        lut_v[pl.ds(48, 16)] = wt1 + wd1

        def chunk_body(ci, carry):
            base = wid * per_w + ci * chunk
            pltpu.sync_copy(attr_hbm.at[pl.ds(base * 2, chunk * 2)], attr_v)

            def group_body(g, c2):
                pairs = g * 32 + iota * 2
                a0 = plsc.load_gather(attr_v, [pairs])
                a1 = plsc.load_gather(attr_v, [pairs + 1])
                cb = (a0 * 2 + a1) * 16
                pos = g * 256 + iota * 16
                for d in range(16):
                    pm = jnp.bitwise_and(iota + d, 15)
                    val = plsc.load_gather(lut_v, [cb + pm])
                    plsc.store_scatter(rows_v, [pos + pm], val)
                return c2

            lax.fori_loop(0, chunk // 16, group_body, 0)
            pltpu.sync_copy(rows_v, out_hbm.at[pl.ds(base * EMB, chunk * EMB)])
            return carry

        lax.fori_loop(0, n_chunks, chunk_body, 0)

    return run


def kernel(edge_attr, W_type, W_dir):
    n_edges = edge_attr.shape[0]
    run = _edge_encode(n_edges)
    out = run(
        edge_attr.astype(jnp.int32).reshape(n_edges * 2),
        W_type.reshape(2 * EMB),
        W_dir.reshape(2 * EMB),
    )
    return out.reshape(n_edges, EMB)


# speedup vs baseline: 1.6594x; 1.0476x over previous
import functools

import jax
import jax.numpy as jnp
from jax import lax
from jax.experimental import pallas as pl
from jax.experimental.pallas import tpu as pltpu
from jax.experimental.pallas import tpu_sc as plsc

EMB = 16
NC = 2
NS = 16
NW = NC * NS


def _edge_encode(n_edges):
    per_w = n_edges // NW
    chunk = 2000
    while per_w % chunk:
        chunk -= 16
    n_chunks = per_w // chunk

    mesh = plsc.VectorSubcoreMesh(core_axis_name="c", subcore_axis_name="s")

    @functools.partial(
        pl.kernel,
        mesh=mesh,
        out_type=jax.ShapeDtypeStruct((n_edges * EMB,), jnp.float32),
        compiler_params=pltpu.CompilerParams(needs_layout_passes=False),
        scratch_types=[
            pltpu.VMEM((chunk * 2,), jnp.int32),
            pltpu.VMEM((chunk * EMB,), jnp.float32),
            pltpu.VMEM((2 * EMB,), jnp.float32),
            pltpu.VMEM((2 * EMB,), jnp.float32),
            pltpu.VMEM((4 * EMB,), jnp.float32),
        ],
    )
    def run(attr_hbm, wt_hbm, wd_hbm, out_hbm, attr_v, rows_v, wt_v, wd_v, lut_v):
        wid = lax.axis_index("s") * NC + lax.axis_index("c")
        iota = lax.iota(jnp.int32, 16)

        pltpu.sync_copy(wt_hbm, wt_v)
        pltpu.sync_copy(wd_hbm, wd_v)
        wt0 = wt_v[pl.ds(0, 16)]
        wt1 = wt_v[pl.ds(16, 16)]
        wd0 = wd_v[pl.ds(0, 16)]
        wd1 = wd_v[pl.ds(16, 16)]
        lut_v[pl.ds(0, 16)] = wt0 + wd0
        lut_v[pl.ds(16, 16)] = wt0 + wd1
        lut_v[pl.ds(32, 16)] = wt1 + wd0
        lut_v[pl.ds(48, 16)] = wt1 + wd1

        def chunk_body(ci, carry):
            base = wid * per_w + ci * chunk
            pltpu.sync_copy(attr_hbm.at[pl.ds(base * 2, chunk * 2)], attr_v)

            def group_body(g, c2):
                pairs = g * 32 + iota * 2
                a0 = plsc.load_gather(attr_v, [pairs])
                a1 = plsc.load_gather(attr_v, [pairs + 1])
                cb = (a0 * 2 + a1) * 16
                pos = g * 256 + iota * 16
                vals = []
                for d in range(16):
                    pm = jnp.bitwise_and(iota + d, 15)
                    vals.append(plsc.load_gather(lut_v, [cb + pm]))
                for d in range(16):
                    pm = jnp.bitwise_and(iota + d, 15)
                    plsc.store_scatter(rows_v, [pos + pm], vals[d])
                return c2

            lax.fori_loop(0, chunk // 16, group_body, 0)
            pltpu.sync_copy(rows_v, out_hbm.at[pl.ds(base * EMB, chunk * EMB)])
            return carry

        lax.fori_loop(0, n_chunks, chunk_body, 0)

    return run


def kernel(edge_attr, W_type, W_dir):
    n_edges = edge_attr.shape[0]
    run = _edge_encode(n_edges)
    out = run(
        edge_attr.astype(jnp.int32).reshape(n_edges * 2),
        W_type.reshape(2 * EMB),
        W_dir.reshape(2 * EMB),
    )
    return out.reshape(n_edges, EMB)
